# baseline (speedup 1.0000x reference)
import jax
import jax.numpy as jnp
from jax.experimental import pallas as pl

_NBLK = 32768
_CH = 2048


def _body(w_ref, b_ref, x_ref, o_ref):
    def chunk(i, carry):
        s = pl.ds(i * _CH, _CH)
        acc = jnp.broadcast_to(b_ref[:], (16, _CH))
        for c in range(32):
            acc = acc + w_ref[:, c : c + 1] * x_ref[0, c : c + 1, s]
        o_ref[0, :, s] = acc
        return carry

    jax.lax.fori_loop(0, _NBLK // _CH, chunk, 0)


def kernel(x, W, b):
    B, C, H, Wd = x.shape
    O = W.shape[0]
    N = H * Wd
    xf = x.reshape(B, C, N)
    b2 = b.reshape(O, 1)
    grid = (B, N // _NBLK)
    out = pl.pallas_call(
        _body,
        grid=grid,
        in_specs=[
            pl.BlockSpec((O, C), lambda i, j: (0, 0)),
            pl.BlockSpec((O, 1), lambda i, j: (0, 0)),
            pl.BlockSpec((1, C, _NBLK), lambda i, j: (i, 0, j)),
        ],
        out_specs=pl.BlockSpec((1, O, _NBLK), lambda i, j: (i, 0, j)),
        out_shape=jax.ShapeDtypeStruct((B, O, N), jnp.float32),
    )(W, b2, xf)
    return out.reshape(B, O, H, Wd)


# speedup vs baseline: 1.6533x; 1.6533x over previous
import jax
import jax.numpy as jnp
from jax.experimental import pallas as pl
from jax.experimental.pallas import tpu as pltpu

_NBLK = 32768


def _body(w_ref, b_ref, x_ref, o_ref):
    o_ref[0] = (
        jnp.dot(w_ref[:], x_ref[0], preferred_element_type=jnp.float32)
        + b_ref[:]
    )


def kernel(x, W, b):
    B, C, H, Wd = x.shape
    O = W.shape[0]
    N = H * Wd
    xf = x.reshape(B, C, N)
    b2 = b.reshape(O, 1)
    grid = (B, N // _NBLK)
    out = pl.pallas_call(
        _body,
        grid=grid,
        in_specs=[
            pl.BlockSpec((O, C), lambda i, j: (0, 0)),
            pl.BlockSpec((O, 1), lambda i, j: (0, 0)),
            pl.BlockSpec((1, C, _NBLK), lambda i, j: (i, 0, j)),
        ],
        out_specs=pl.BlockSpec((1, O, _NBLK), lambda i, j: (i, 0, j)),
        out_shape=jax.ShapeDtypeStruct((B, O, N), jnp.float32),
        compiler_params=pltpu.CompilerParams(
            dimension_semantics=("parallel", "parallel"),
        ),
    )(W, b2, xf)
    return out.reshape(B, O, H, Wd)
